# baseline (speedup 1.0000x reference)
import functools
import math

import jax
import jax.numpy as jnp
from jax import lax
from jax.experimental import pallas as pl
from jax.experimental.pallas import tpu as pltpu
from jax.experimental.pallas import tpu_sc as plsc

N_FFT = 2048
HOP = 512
GAMMA = 2 * math.pi * ((-(N_FFT ** 2) / (8 * math.log(0.01))) ** 0.5) ** 2
ABSTOL = 1e-10
N = N_FFT // 2 + 1
NV = 65
NPAD = NV * 16
NIN = N + 32
VB = NIN + 32

INV4F = 1.0 / (4.0 * (GAMMA / (2 * HOP * N_FFT)))
LINC = 2 * math.pi * HOP / N_FFT
LN2 = 0.6931471805599453
SQRT2 = 1.4142135623730951
NEG = -3.4e38

_GDN = lax.GatherDimensionNumbers(
    offset_dims=(), collapsed_slice_dims=(0,), start_index_map=(0,))


def _gat(x, idx):
    return lax.gather(x, idx[:, None], _GDN, (1,),
                      mode=lax.GatherScatterMode.PROMISE_IN_BOUNDS)


def _log16(xr):
    bits = plsc.bitcast(xr, jnp.int32)
    e = lax.shift_right_logical(bits, 23) - 127
    mbits = (bits & 0x007FFFFF) | 0x3F800000
    m = plsc.bitcast(mbits, jnp.float32)
    big = m > SQRT2
    m = jnp.where(big, m * 0.5, m)
    e = e + jnp.where(big, 1, 0)
    t = m - 1.0
    s = t / (2.0 + t)
    z = s * s
    w = jnp.float32(1.0 / 7.0)
    w = jnp.float32(1.0 / 5.0) + z * w
    w = jnp.float32(1.0 / 3.0) + z * w
    return e.astype(jnp.float32) * jnp.float32(LN2) + (2.0 * s + (2.0 * s) * (z * w))


def _lex_scan_steps(m, bi, f, sidx, sinr):
    for k in range(4):
        ms = jnp.where(sinr[k], _gat(m, sidx[k]), NEG)
        is_ = _gat(bi, sidx[k])
        fs = jnp.where(sinr[k], _gat(f, sidx[k]), 0)
        take = (f == 0) & ((ms > m) | ((ms == m) & (is_ < bi)))
        m = jnp.where(take, ms, m)
        bi = jnp.where(take, is_, bi)
        f = f | fs
    return m, bi, f


def _lex_carry(m, bi, f, mc, ic):
    take = (f == 0) & ((mc > m) | ((mc == m) & (ic < bi)))
    return jnp.where(take, mc, m), jnp.where(take, ic, bi)


def _body(x_hbm, out_hbm, xv, vbuf, cc, mf, jf, ph):
    cid = lax.axis_index("c")
    sid = lax.axis_index("s")

    @pl.when((cid == 0) & (sid == 0))
    def _():
        pltpu.sync_copy(x_hbm, xv.at[pl.ds(0, N)])
        xv[pl.ds(N, 16)] = jnp.full((16,), 0.5, jnp.float32)
        xv[pl.ds(N + 16, 16)] = jnp.full((16,), 0.5, jnp.float32)
        lane = lax.iota(jnp.int32, 16)
        i0 = jnp.zeros((16,), jnp.int32)
        i15 = jnp.full((16,), 15, jnp.int32)
        lm1 = jnp.maximum(lane - 1, 0)
        lm2 = jnp.maximum(lane - 2, 0)
        lp1 = jnp.minimum(lane + 1, 15)
        l14 = jnp.minimum(lane + 14, 15)
        sidx = [jnp.maximum(lane - d, 0) for d in (1, 2, 4, 8)]
        sinr = [lane >= d for d in (1, 2, 4, 8)]
        negv = jnp.full((16,), -1.0, jnp.float32)
        vbuf[pl.ds(0, 16)] = negv
        vbuf[pl.ds(NIN + 16, 16)] = negv
        v0_init = _log16(xv[pl.ds(0, 16)])
        vbuf[pl.ds(16, 16)] = v0_init

        def p1_step(b, carry):
            cs, mc, ic, vm1, vp0 = carry
            base = 16 + b * 16
            vnx = _log16(xv[pl.ds(b * 16 + 16, 16)])
            vbuf[pl.ds(base + 16, 16)] = vnx
            vm1v = jnp.where(lane >= 1, _gat(vp0, lm1), _gat(vm1, i15))
            vm2v = jnp.where(lane >= 2, _gat(vp0, lm2), _gat(vm1, l14))
            vp1v = jnp.where(lane <= 14, _gat(vp0, lp1), _gat(vnx, i0))
            jj = lane + b * 16
            ja = jj - 1
            ga = jnp.float32(INV4F) * (vp0 - vm2v) + jnp.float32(LINC) * ja.astype(jnp.float32)
            ga = jnp.where((ja >= 1) & (ja <= N - 2), ga, 0.0)
            gb = jnp.float32(INV4F) * (vp1v - vm1v) + jnp.float32(LINC) * jj.astype(jnp.float32)
            gb = jnp.where((jj >= 1) & (jj <= N - 2), gb, 0.0)
            dstep = jnp.where(jj >= 1, 0.5 * (ga + gb), 0.0)
            cvec = plsc.cumsum(dstep) + cs
            cc[pl.ds(b * 16, 16)] = cvec
            act = vp0 > ABSTOL
            m = jnp.where(act, vp0, NEG)
            f = (jnp.logical_not(act) | (vm1v <= ABSTOL)).astype(jnp.int32)
            m, bi, f = _lex_scan_steps(m, jj, f, sidx, sinr)
            m, bi = _lex_carry(m, bi, f, mc, ic)
            mf[pl.ds(b * 16, 16)] = m
            jf[pl.ds(b * 16, 16)] = bi
            return _gat(cvec, i15), _gat(m, i15), _gat(bi, i15), vp0, vnx

        def p1(t, carry):
            return p1_step(2 * t + 1, p1_step(2 * t, carry))

        c1 = lax.fori_loop(0, (NV - 1) // 2, p1,
                           (jnp.zeros((16,), jnp.float32),
                            jnp.full((16,), NEG, jnp.float32),
                            jnp.zeros((16,), jnp.int32),
                            negv, v0_init))
        p1_step(NV - 1, c1)

        def p2_step(bb, carry):
            mc, ic = carry
            base = 16 + bb * 16
            v0 = vbuf[pl.ds(base, 16)]
            vp1 = vbuf[pl.ds(base + 1, 16)]
            act = v0 > ABSTOL
            jj = lane + bb * 16
            mr = lax.rev(jnp.where(act, v0, NEG), (0,))
            br = lax.rev(jj, (0,))
            fr = lax.rev((jnp.logical_not(act) | (vp1 <= ABSTOL)).astype(jnp.int32), (0,))
            mr, br, fr = _lex_scan_steps(mr, br, fr, sidx, sinr)
            mr, br = _lex_carry(mr, br, fr, mc, ic)
            nmc = _gat(mr, i15)
            nic = _gat(br, i15)
            mb = lax.rev(mr, (0,))
            jb = lax.rev(br, (0,))
            mfv = mf[pl.ds(bb * 16, 16)]
            jfv = jf[pl.ds(bb * 16, 16)]
            take = (mb > mfv) | ((mb == mfv) & (jb < jfv))
            seed = jnp.where(take, jb, jfv)
            cs = plsc.load_gather(cc, [seed])
            cv = cc[pl.ds(bb * 16, 16)]
            ph[pl.ds(bb * 16, 16)] = jnp.where(act, cv - cs, 0.0)
            return nmc, nic

        def p2(t, carry):
            return p2_step(NV - 2 - 2 * t, p2_step(NV - 1 - 2 * t, carry))

        c2 = lax.fori_loop(0, (NV - 1) // 2, p2,
                           (jnp.full((16,), NEG, jnp.float32),
                            jnp.zeros((16,), jnp.int32)))
        p2_step(0, c2)

        pltpu.sync_copy(ph.at[pl.ds(0, N)], out_hbm)


_pghi_sc = functools.partial(
    pl.kernel,
    out_type=jax.ShapeDtypeStruct((N,), jnp.float32),
    mesh=plsc.VectorSubcoreMesh(core_axis_name="c", subcore_axis_name="s",
                                num_cores=1),
    compiler_params=pltpu.CompilerParams(needs_layout_passes=False),
    scratch_types=[
        pltpu.VMEM((NIN,), jnp.float32),
        pltpu.VMEM((VB,), jnp.float32),
        pltpu.VMEM((NPAD,), jnp.float32),
        pltpu.VMEM((NPAD,), jnp.float32),
        pltpu.VMEM((NPAD,), jnp.int32),
        pltpu.VMEM((NPAD,), jnp.float32),
    ],
)(_body)


def kernel(x, mag_buffer):
    return _pghi_sc(x.reshape(N)).reshape(x.shape)


# speedup vs baseline: 1.0484x; 1.0484x over previous
import functools
import math

import jax
import jax.numpy as jnp
from jax import lax
from jax.experimental import pallas as pl
from jax.experimental.pallas import tpu as pltpu
from jax.experimental.pallas import tpu_sc as plsc

N_FFT = 2048
HOP = 512
GAMMA = 2 * math.pi * ((-(N_FFT ** 2) / (8 * math.log(0.01))) ** 0.5) ** 2
ABSTOL = 1e-10
N = N_FFT // 2 + 1
NV = 65
NPAD = NV * 16
NIN = N + 32
VB = NIN + 32

INV4F = 1.0 / (4.0 * (GAMMA / (2 * HOP * N_FFT)))
LINC = 2 * math.pi * HOP / N_FFT
LN2 = 0.6931471805599453
SQRT2 = 1.4142135623730951
NEG = -3.4e38

_GDN = lax.GatherDimensionNumbers(
    offset_dims=(), collapsed_slice_dims=(0,), start_index_map=(0,))


def _gat(x, idx):
    return lax.gather(x, idx[:, None], _GDN, (1,),
                      mode=lax.GatherScatterMode.PROMISE_IN_BOUNDS)


def _log16(xr):
    bits = plsc.bitcast(xr, jnp.int32)
    e = lax.shift_right_logical(bits, 23) - 127
    mbits = (bits & 0x007FFFFF) | 0x3F800000
    m = plsc.bitcast(mbits, jnp.float32)
    big = m > SQRT2
    m = jnp.where(big, m * 0.5, m)
    e = e + jnp.where(big, 1, 0)
    t = m - 1.0
    s = t / (2.0 + t)
    z = s * s
    w = jnp.float32(1.0 / 7.0)
    w = jnp.float32(1.0 / 5.0) + z * w
    w = jnp.float32(1.0 / 3.0) + z * w
    return e.astype(jnp.float32) * jnp.float32(LN2) + (2.0 * s + (2.0 * s) * (z * w))


def _lex_scan_steps(m, bi, f, sidx, sinr):
    for k in range(4):
        ms = jnp.where(sinr[k], _gat(m, sidx[k]), NEG)
        is_ = _gat(bi, sidx[k])
        fs = jnp.where(sinr[k], _gat(f, sidx[k]), 0)
        take = (f == 0) & ((ms > m) | ((ms == m) & (is_ < bi)))
        m = jnp.where(take, ms, m)
        bi = jnp.where(take, is_, bi)
        f = f | fs
    return m, bi, f


def _lex_carry(m, bi, f, mc, ic):
    take = (f == 0) & ((mc > m) | ((mc == m) & (ic < bi)))
    return jnp.where(take, mc, m), jnp.where(take, ic, bi)


def _body(x_hbm, out_hbm, xv, vbuf, cc, mf, jf, ph):
    cid = lax.axis_index("c")
    sid = lax.axis_index("s")

    @pl.when((cid == 0) & (sid == 0))
    def _():
        pltpu.sync_copy(x_hbm, xv.at[pl.ds(0, N)])
        xv[pl.ds(N, 16)] = jnp.full((16,), 0.5, jnp.float32)
        xv[pl.ds(N + 16, 16)] = jnp.full((16,), 0.5, jnp.float32)
        lane = lax.iota(jnp.int32, 16)
        i0 = jnp.zeros((16,), jnp.int32)
        i15 = jnp.full((16,), 15, jnp.int32)
        lm1 = jnp.maximum(lane - 1, 0)
        lm2 = jnp.maximum(lane - 2, 0)
        lp1 = jnp.minimum(lane + 1, 15)
        l14 = jnp.minimum(lane + 14, 15)
        sidx = [jnp.maximum(lane - d, 0) for d in (1, 2, 4, 8)]
        sinr = [lane >= d for d in (1, 2, 4, 8)]
        negv = jnp.full((16,), -1.0, jnp.float32)
        vbuf[pl.ds(0, 16)] = negv
        vbuf[pl.ds(NIN + 16, 16)] = negv
        v0_init = _log16(xv[pl.ds(0, 16)])
        vbuf[pl.ds(16, 16)] = v0_init

        def p1_step(b, carry):
            cs, mc, ic, vm1, vp0 = carry
            base = 16 + b * 16
            vnx = _log16(xv[pl.ds(b * 16 + 16, 16)])
            vbuf[pl.ds(base + 16, 16)] = vnx
            vm1v = jnp.where(lane >= 1, _gat(vp0, lm1), _gat(vm1, i15))
            vm2v = jnp.where(lane >= 2, _gat(vp0, lm2), _gat(vm1, l14))
            vp1v = jnp.where(lane <= 14, _gat(vp0, lp1), _gat(vnx, i0))
            jj = lane + b * 16
            ja = jj - 1
            ga = jnp.float32(INV4F) * (vp0 - vm2v) + jnp.float32(LINC) * ja.astype(jnp.float32)
            ga = jnp.where((ja >= 1) & (ja <= N - 2), ga, 0.0)
            gb = jnp.float32(INV4F) * (vp1v - vm1v) + jnp.float32(LINC) * jj.astype(jnp.float32)
            gb = jnp.where((jj >= 1) & (jj <= N - 2), gb, 0.0)
            dstep = jnp.where(jj >= 1, 0.5 * (ga + gb), 0.0)
            cvec = plsc.cumsum(dstep) + cs
            cc[pl.ds(b * 16, 16)] = cvec
            act = vp0 > ABSTOL
            m = jnp.where(act, vp0, NEG)
            f = (jnp.logical_not(act) | (vm1v <= ABSTOL)).astype(jnp.int32)
            m, bi, f = _lex_scan_steps(m, jj, f, sidx, sinr)
            m, bi = _lex_carry(m, bi, f, mc, ic)
            mf[pl.ds(b * 16, 16)] = m
            jf[pl.ds(b * 16, 16)] = bi
            return _gat(cvec, i15), _gat(m, i15), _gat(bi, i15), vp0, vnx

        plsc.parallel_loop(0, NV, 1, unroll=2, carry=(
            jnp.zeros((16,), jnp.float32),
            jnp.full((16,), NEG, jnp.float32),
            jnp.zeros((16,), jnp.int32),
            negv, v0_init))(p1_step)

        def p2_step(t, carry):
            bb = NV - 1 - t
            mc, ic = carry
            base = 16 + bb * 16
            v0 = vbuf[pl.ds(base, 16)]
            vp1 = vbuf[pl.ds(base + 1, 16)]
            act = v0 > ABSTOL
            jj = lane + bb * 16
            mr = lax.rev(jnp.where(act, v0, NEG), (0,))
            br = lax.rev(jj, (0,))
            fr = lax.rev((jnp.logical_not(act) | (vp1 <= ABSTOL)).astype(jnp.int32), (0,))
            mr, br, fr = _lex_scan_steps(mr, br, fr, sidx, sinr)
            mr, br = _lex_carry(mr, br, fr, mc, ic)
            nmc = _gat(mr, i15)
            nic = _gat(br, i15)
            mb = lax.rev(mr, (0,))
            jb = lax.rev(br, (0,))
            mfv = mf[pl.ds(bb * 16, 16)]
            jfv = jf[pl.ds(bb * 16, 16)]
            take = (mb > mfv) | ((mb == mfv) & (jb < jfv))
            seed = jnp.where(take, jb, jfv)
            cs = plsc.load_gather(cc, [seed])
            cv = cc[pl.ds(bb * 16, 16)]
            ph[pl.ds(bb * 16, 16)] = jnp.where(act, cv - cs, 0.0)
            return nmc, nic

        plsc.parallel_loop(0, NV, 1, unroll=2, carry=(
            jnp.full((16,), NEG, jnp.float32),
            jnp.zeros((16,), jnp.int32)))(p2_step)

        pltpu.sync_copy(ph.at[pl.ds(0, N)], out_hbm)


_pghi_sc = functools.partial(
    pl.kernel,
    out_type=jax.ShapeDtypeStruct((N,), jnp.float32),
    mesh=plsc.VectorSubcoreMesh(core_axis_name="c", subcore_axis_name="s",
                                num_cores=1),
    compiler_params=pltpu.CompilerParams(needs_layout_passes=False),
    scratch_types=[
        pltpu.VMEM((NIN,), jnp.float32),
        pltpu.VMEM((VB,), jnp.float32),
        pltpu.VMEM((NPAD,), jnp.float32),
        pltpu.VMEM((NPAD,), jnp.float32),
        pltpu.VMEM((NPAD,), jnp.int32),
        pltpu.VMEM((NPAD,), jnp.float32),
    ],
)(_body)


def kernel(x, mag_buffer):
    return _pghi_sc(x.reshape(N)).reshape(x.shape)
